# baseline (speedup 1.0000x reference)
import functools

import jax
import jax.numpy as jnp
from jax import lax
from jax.experimental import pallas as pl
from jax.experimental.pallas import tpu as pltpu
from jax.experimental.pallas import tpu_sc as plsc

_NC = 2
_NS = 16
_NW = _NC * _NS
_BC = 512


@jax.jit
def _embed_native(idx_t, tab_t):
    L, B = idx_t.shape
    D, V = tab_t.shape
    n_bch = B // _BC
    n_units = L * n_bch
    per_w = n_units // _NW
    mesh = plsc.VectorSubcoreMesh(
        core_axis_name="c", subcore_axis_name="s",
        num_cores=_NC, num_subcores=_NS)

    @functools.partial(
        pl.kernel,
        out_type=jax.ShapeDtypeStruct((L, D, B), jnp.float32),
        mesh=mesh,
        scratch_types=[
            pltpu.VMEM((_BC,), jnp.int32),
            pltpu.VMEM((_BC,), jnp.int32),
            pltpu.VMEM((D, _BC), jnp.float32),
            pltpu.VMEM((D, _BC), jnp.float32),
            pltpu.SemaphoreType.DMA,
            pltpu.SemaphoreType.DMA,
            pltpu.SemaphoreType.DMA,
            pltpu.SemaphoreType.DMA,
            pltpu.SemaphoreType.DMA,
            pltpu.SemaphoreType.DMA,
        ],
        compiler_params=pltpu.CompilerParams(
            use_tc_tiling_on_sc=False, needs_layout_passes=False),
    )
    def k(idx_hbm, tab_hbm, out_hbm,
          idx0, idx1, gbuf0, gbuf1,
          is0, is1, gs0, gs1, os0, os1):
        wid = lax.axis_index("s") * _NC + lax.axis_index("c")
        u0 = wid * per_w
        idxv = (idx0, idx1)
        gbufv = (gbuf0, gbuf1)
        isem = (is0, is1)
        gsem = (gs0, gs1)
        osem = (os0, os1)

        def unit_lb(j):
            u = u0 + j
            return u >> 3, u & (n_bch - 1)

        def idx_src(j):
            l, bch = unit_lb(j)
            return idx_hbm.at[l, pl.ds(bch * _BC, _BC)]

        def out_dst(j):
            l, bch = unit_lb(j)
            return out_hbm.at[l, :, pl.ds(bch * _BC, _BC)]

        def start_idx(j, b):
            pltpu.async_copy(idx_src(j), idxv[b], isem[b])

        def wait_idx(j, b):
            pltpu.make_async_copy(idx_src(j), idxv[b], isem[b]).wait()

        def fire_gathers(b):
            @pl.loop(0, D, unroll=8)
            def _(d):
                pltpu.async_copy(tab_hbm.at[d].at[idxv[b]],
                                 gbufv[b].at[d], gsem[b])

        def drain_gathers(b):
            @pl.loop(0, D, unroll=8)
            def _(d):
                pltpu.make_async_copy(tab_hbm.at[d].at[idxv[b]],
                                      gbufv[b].at[d], gsem[b]).wait()

        def start_out(j, b):
            pltpu.async_copy(gbufv[b], out_dst(j), osem[b])

        def wait_out(j, b):
            pltpu.make_async_copy(gbufv[b], out_dst(j), osem[b]).wait()

        start_idx(0, 0)
        start_idx(1, 1)
        wait_idx(0, 0)
        fire_gathers(0)

        @pl.loop(0, per_w, step=2)
        def _(j0):
            for t in range(2):
                j = j0 + t
                b = t
                ob = 1 - t

                @pl.when(j + 1 < per_w)
                def _():
                    wait_idx(j + 1, ob)

                    @pl.when(j >= 1)
                    def _():
                        wait_out(j - 1, ob)

                    fire_gathers(ob)

                drain_gathers(b)

                @pl.when(j + 2 < per_w)
                def _():
                    start_idx(j + 2, b)

                start_out(j, b)

        wait_out(per_w - 2, 0)
        wait_out(per_w - 1, 1)

    return k(idx_t, tab_t)


def kernel(indices, table):
    out_t = _embed_native(indices.T, table.T)
    return jnp.transpose(out_t, (2, 0, 1))


# speedup vs baseline: 5.7620x; 5.7620x over previous
import functools

import jax
import jax.numpy as jnp
from jax import lax
from jax.experimental import pallas as pl
from jax.experimental.pallas import tpu as pltpu
from jax.experimental.pallas import tpu_sc as plsc

_NC = 2
_NS = 16
_NW = _NC * _NS
_CH = 320
_NB = 4


@jax.jit
def _embed_rows(idx_flat, table):
    N = idx_flat.shape[0]
    V, D = table.shape
    per_rows = N // _NW
    n_ch = per_rows // _CH
    mesh = plsc.VectorSubcoreMesh(
        core_axis_name="c", subcore_axis_name="s",
        num_cores=_NC, num_subcores=_NS)

    @functools.partial(
        pl.kernel,
        out_type=jax.ShapeDtypeStruct((N, D), jnp.float32),
        mesh=mesh,
        scratch_types=[
            pltpu.VMEM((per_rows,), jnp.int32),
            pltpu.VMEM((_CH, D), jnp.float32),
            pltpu.VMEM((_CH, D), jnp.float32),
            pltpu.VMEM((_CH, D), jnp.float32),
            pltpu.VMEM((_CH, D), jnp.float32),
            pltpu.SemaphoreType.DMA,
            pltpu.SemaphoreType.DMA,
            pltpu.SemaphoreType.DMA,
            pltpu.SemaphoreType.DMA,
            pltpu.SemaphoreType.DMA,
            pltpu.SemaphoreType.DMA,
            pltpu.SemaphoreType.DMA,
            pltpu.SemaphoreType.DMA,
            pltpu.SemaphoreType.DMA,
        ],
        compiler_params=pltpu.CompilerParams(
            use_tc_tiling_on_sc=False, needs_layout_passes=False),
    )
    def k(idx_hbm, tab_hbm, out_hbm,
          idx1d, rows0, rows1, rows2, rows3,
          isem, g0, g1, g2, g3, o0, o1, o2, o3):
        wid = lax.axis_index("s") * _NC + lax.axis_index("c")
        base = wid * per_rows
        rowsv = (rows0, rows1, rows2, rows3)
        gsem = (g0, g1, g2, g3)
        osem = (o0, o1, o2, o3)

        idx_all_src = idx_hbm.at[pl.ds(base, per_rows)]
        pltpu.async_copy(idx_all_src, idx1d, isem)
        pltpu.make_async_copy(idx_all_src, idx1d, isem).wait()

        def idx_ref(u):
            return idx1d.at[pl.ds(u * _CH, _CH)]

        def out_dst(u):
            return out_hbm.at[pl.ds(base + u * _CH, _CH)]

        def start_g(u, b):
            pltpu.async_copy(tab_hbm.at[idx_ref(u)], rowsv[b], gsem[b])

        def wait_g(u, b):
            pltpu.make_async_copy(tab_hbm.at[idx_ref(u)], rowsv[b],
                                  gsem[b]).wait()

        def start_o(u, b):
            pltpu.async_copy(rowsv[b], out_dst(u), osem[b])

        def wait_o(u, b):
            pltpu.make_async_copy(rowsv[b], out_dst(u), osem[b]).wait()

        start_g(0, 0)
        start_g(1, 1)
        start_g(2, 2)

        @pl.loop(0, n_ch, step=_NB)
        def _(u0):
            for t in range(_NB):
                u = u0 + t
                b = t
                b3 = (t + 3) % _NB
                wait_g(u, b)
                start_o(u, b)

                @pl.when(u + 3 < n_ch)
                def _():
                    @pl.when(u >= 1)
                    def _():
                        wait_o(u - 1, b3)

                    start_g(u + 3, b3)

        wait_o(n_ch - 4, 0)
        wait_o(n_ch - 3, 1)
        wait_o(n_ch - 2, 2)
        wait_o(n_ch - 1, 3)

    return k(idx_flat, table)


def kernel(indices, table):
    B, L = indices.shape
    D = table.shape[1]
    out_flat = _embed_rows(indices.reshape(-1), table)
    return out_flat.reshape(B, L, D)
